# initial kernel scaffold (unmeasured)
import jax
import jax.numpy as jnp
from jax import lax
from jax.experimental import pallas as pl
from jax.experimental.pallas import tpu as pltpu


def kernel(
    x,
):
    def body(*refs):
        pass

    out_shape = jax.ShapeDtypeStruct(..., jnp.float32)
    return pl.pallas_call(body, out_shape=out_shape)(...)


# baseline (device time: 15342 ns/iter reference)
import jax
import jax.numpy as jnp
from jax import lax
from jax.experimental import pallas as pl
from jax.experimental.pallas import tpu as pltpu

N_DEV = 8
N_ROUNDS = 3


def kernel(x):
    m, n = x.shape

    def body(x_ref, out_ref, recv_bufs, send_sems, recv_sems):
        my = lax.axis_index("i")

        barrier_sem = pltpu.get_barrier_semaphore()
        for r in range(N_ROUNDS):
            partner = my ^ (1 << r)
            pl.semaphore_signal(
                barrier_sem, inc=1,
                device_id=(partner,), device_id_type=pl.DeviceIdType.MESH,
            )
        pl.semaphore_wait(barrier_sem, N_ROUNDS)

        out_ref[...] = x_ref[...].astype(out_ref.dtype)

        for r in range(N_ROUNDS):
            partner = my ^ (1 << r)
            rdma = pltpu.make_async_remote_copy(
                src_ref=out_ref,
                dst_ref=recv_bufs.at[r],
                send_sem=send_sems.at[r],
                recv_sem=recv_sems.at[r],
                device_id=(partner,),
                device_id_type=pl.DeviceIdType.MESH,
            )
            rdma.start()
            rdma.wait()
            out_ref[...] = out_ref[...] + recv_bufs[r]

    return pl.pallas_call(
        body,
        out_shape=jax.ShapeDtypeStruct((m, n), jnp.bfloat16),
        in_specs=[pl.BlockSpec(memory_space=pltpu.VMEM)],
        out_specs=pl.BlockSpec(memory_space=pltpu.VMEM),
        scratch_shapes=[
            pltpu.VMEM((N_ROUNDS, m, n), jnp.bfloat16),
            pltpu.SemaphoreType.DMA((N_ROUNDS,)),
            pltpu.SemaphoreType.DMA((N_ROUNDS,)),
        ],
        compiler_params=pltpu.CompilerParams(collective_id=0),
    )(x)


# device time: 12745 ns/iter; 1.2038x vs baseline; 1.2038x over previous
import jax
import jax.numpy as jnp
from jax import lax
from jax.experimental import pallas as pl
from jax.experimental.pallas import tpu as pltpu

N_DEV = 8


def kernel(x):
    m, n = x.shape
    chunk = m // N_DEV

    def body(x_ref, out_ref, xbf, rs_bufs, mine,
             rs_send_sem, rs_recv_sem, ag_send_sem, ag_recv_sem):
        my = lax.axis_index("i")

        barrier_sem = pltpu.get_barrier_semaphore()
        for k in range(1, N_DEV):
            pl.semaphore_signal(
                barrier_sem, inc=1,
                device_id=((my + k) % N_DEV,),
                device_id_type=pl.DeviceIdType.MESH,
            )
        pl.semaphore_wait(barrier_sem, N_DEV - 1)

        xbf[...] = x_ref[...].astype(jnp.bfloat16)

        rs_rdmas = []
        for k in range(1, N_DEV):
            dst = (my + k) % N_DEV
            rdma = pltpu.make_async_remote_copy(
                src_ref=xbf.at[pl.ds(dst * chunk, chunk), :],
                dst_ref=rs_bufs.at[pl.ds(my * chunk, chunk), :],
                send_sem=rs_send_sem,
                recv_sem=rs_recv_sem,
                device_id=(dst,),
                device_id_type=pl.DeviceIdType.MESH,
            )
            rdma.start()
            rs_rdmas.append(rdma)
        rs_bufs[pl.ds(my * chunk, chunk), :] = xbf[pl.ds(my * chunk, chunk), :]

        for rdma in rs_rdmas:
            rdma.wait_recv()
        acc = rs_bufs[0:chunk, :]
        for s in range(1, N_DEV):
            acc = acc + rs_bufs[s * chunk:(s + 1) * chunk, :]
        mine[...] = acc
        out_ref[pl.ds(my * chunk, chunk), :] = acc

        ag_rdmas = []
        for k in range(1, N_DEV):
            dst = (my + k) % N_DEV
            rdma = pltpu.make_async_remote_copy(
                src_ref=mine,
                dst_ref=out_ref.at[pl.ds(my * chunk, chunk), :],
                send_sem=ag_send_sem,
                recv_sem=ag_recv_sem,
                device_id=(dst,),
                device_id_type=pl.DeviceIdType.MESH,
            )
            rdma.start()
            ag_rdmas.append(rdma)
        for rdma in ag_rdmas:
            rdma.wait_recv()

        for rdma in rs_rdmas:
            rdma.wait_send()
        for rdma in ag_rdmas:
            rdma.wait_send()

    return pl.pallas_call(
        body,
        out_shape=jax.ShapeDtypeStruct((m, n), jnp.bfloat16),
        in_specs=[pl.BlockSpec(memory_space=pltpu.VMEM)],
        out_specs=pl.BlockSpec(memory_space=pltpu.VMEM),
        scratch_shapes=[
            pltpu.VMEM((m, n), jnp.bfloat16),
            pltpu.VMEM((N_DEV * chunk, n), jnp.bfloat16),
            pltpu.VMEM((chunk, n), jnp.bfloat16),
            pltpu.SemaphoreType.DMA,
            pltpu.SemaphoreType.DMA,
            pltpu.SemaphoreType.DMA,
            pltpu.SemaphoreType.DMA,
        ],
        compiler_params=pltpu.CompilerParams(collective_id=0),
    )(x)


# device time: 9955 ns/iter; 1.5411x vs baseline; 1.2803x over previous
import jax
import jax.numpy as jnp
from jax import lax
from jax.experimental import pallas as pl
from jax.experimental.pallas import tpu as pltpu

N_DEV = 8

_FARTHEST_FIRST = (6, 5, 7, 2, 1, 3, 4)


def kernel(x):
    m, n = x.shape
    chunk = m // N_DEV

    def body(x_ref, out_ref, xbf, rs_bufs, mine,
             rs_send_sem, rs_recv_sem, ag_send_sem, ag_recv_sem):
        my = lax.axis_index("i")

        barrier_sem = pltpu.get_barrier_semaphore()
        for k in range(1, N_DEV):
            pl.semaphore_signal(
                barrier_sem, inc=1,
                device_id=((my + k) % N_DEV,),
                device_id_type=pl.DeviceIdType.MESH,
            )
        xbf[...] = x_ref[...].astype(jnp.bfloat16)
        rs_bufs[pl.ds(my * chunk, chunk), :] = xbf[pl.ds(my * chunk, chunk), :]
        pl.semaphore_wait(barrier_sem, N_DEV - 1)

        rs_rdmas = []
        for mask in _FARTHEST_FIRST:
            dst = my ^ mask
            rdma = pltpu.make_async_remote_copy(
                src_ref=xbf.at[pl.ds(dst * chunk, chunk), :],
                dst_ref=rs_bufs.at[pl.ds(my * chunk, chunk), :],
                send_sem=rs_send_sem,
                recv_sem=rs_recv_sem,
                device_id=(dst,),
                device_id_type=pl.DeviceIdType.MESH,
            )
            rdma.start()
            rs_rdmas.append(rdma)

        for rdma in rs_rdmas:
            rdma.wait_recv()
        acc = rs_bufs[0:chunk, :]
        for s in range(1, N_DEV):
            acc = acc + rs_bufs[s * chunk:(s + 1) * chunk, :]
        mine[...] = acc
        out_ref[pl.ds(my * chunk, chunk), :] = acc

        ag_rdmas = []
        for mask in _FARTHEST_FIRST:
            dst = my ^ mask
            rdma = pltpu.make_async_remote_copy(
                src_ref=mine,
                dst_ref=out_ref.at[pl.ds(my * chunk, chunk), :],
                send_sem=ag_send_sem,
                recv_sem=ag_recv_sem,
                device_id=(dst,),
                device_id_type=pl.DeviceIdType.MESH,
            )
            rdma.start()
            ag_rdmas.append(rdma)
        for rdma in ag_rdmas:
            rdma.wait_recv()

        for rdma in rs_rdmas:
            rdma.wait_send()
        for rdma in ag_rdmas:
            rdma.wait_send()

    return pl.pallas_call(
        body,
        out_shape=jax.ShapeDtypeStruct((m, n), jnp.bfloat16),
        in_specs=[pl.BlockSpec(memory_space=pltpu.VMEM)],
        out_specs=pl.BlockSpec(memory_space=pltpu.VMEM),
        scratch_shapes=[
            pltpu.VMEM((m, n), jnp.bfloat16),
            pltpu.VMEM((N_DEV * chunk, n), jnp.bfloat16),
            pltpu.VMEM((chunk, n), jnp.bfloat16),
            pltpu.SemaphoreType.DMA,
            pltpu.SemaphoreType.DMA,
            pltpu.SemaphoreType.DMA,
            pltpu.SemaphoreType.DMA,
        ],
        compiler_params=pltpu.CompilerParams(collective_id=0),
    )(x)
